# Initial kernel scaffold; baseline (speedup 1.0000x reference)
import jax
import jax.numpy as jnp
from jax.experimental import pallas as pl


def kernel(x, adj_row, adj_col, inc_node, inc_edge, W1, W2, W3, W4):
    raise NotImplementedError("write your pallas kernel here")


# speedup vs baseline: 2.9068x; 2.9068x over previous
import functools

import jax
import jax.numpy as jnp
from jax import lax
from jax.experimental import pallas as pl
from jax.experimental.pallas import tpu as pltpu
from jax.experimental.pallas import tpu_sc as plsc

N = 10000
D = 128
E1 = 160000
NNZ = 320000

NC = 2
NS = 16
NT = NC * NS
CH = 128

NCH = 79
NNZ_PAD = NT * NCH * CH

NACC = NS * 640
TRASH = N + 16

EB = 8192
NB = 20
EP = EB * NB
ECH = 8

_f32 = jnp.float32
_i32 = jnp.int32


def _pad_to(a, n, fill):
    return jnp.concatenate([a, jnp.full((n - a.shape[0],), fill, a.dtype)])


def _mm2_body(x_ref, w1_ref, w2_ref, y1_ref, y2_ref):
    xb = x_ref[...]
    y1_ref[...] = jnp.dot(xb, w1_ref[...], preferred_element_type=_f32)
    y2_ref[...] = jnp.dot(xb, w2_ref[...], preferred_element_type=_f32)


def _tc_mm2(x, w1, w2):
    bm = 1000
    grid = (N // bm,)
    return pl.pallas_call(
        _mm2_body,
        grid=grid,
        in_specs=[
            pl.BlockSpec((bm, D), lambda i: (i, 0)),
            pl.BlockSpec((D, D), lambda i: (0, 0)),
            pl.BlockSpec((D, D), lambda i: (0, 0)),
        ],
        out_specs=[pl.BlockSpec((bm, D), lambda i: (i, 0))] * 2,
        out_shape=[jax.ShapeDtypeStruct((N, D), _f32)] * 2,
    )(x, w1, w2)


def _sig2mm_body(p_ref, w_ref, o_ref):
    s = jax.nn.sigmoid(p_ref[0] + p_ref[1])
    o_ref[...] = jnp.dot(s, w_ref[...], preferred_element_type=_f32)


def _tc_sig2mm(p, w):
    bm = 1000
    return pl.pallas_call(
        _sig2mm_body,
        grid=(N // bm,),
        in_specs=[
            pl.BlockSpec((2, bm, D), lambda i: (0, i, 0)),
            pl.BlockSpec((D, D), lambda i: (0, 0)),
        ],
        out_specs=pl.BlockSpec((bm, D), lambda i: (i, 0)),
        out_shape=jax.ShapeDtypeStruct((N, D), _f32),
    )(p, w)


def _sigmm_body(e_ref, w_ref, o_ref):
    o_ref[...] = jnp.dot(jax.nn.sigmoid(e_ref[...]), w_ref[...],
                         preferred_element_type=_f32)


def _tc_sigmm(e, w):
    bm = 1024
    return pl.pallas_call(
        _sigmm_body,
        grid=(EP // bm,),
        in_specs=[
            pl.BlockSpec((bm, D), lambda i: (i, 0)),
            pl.BlockSpec((D, D), lambda i: (0, 0)),
        ],
        out_specs=pl.BlockSpec((bm, D), lambda i: (i, 0)),
        out_shape=jax.ShapeDtypeStruct((EP, D), _f32),
    )(e, w)


def _merge_body(pb_ref, pc_ref, o_ref):
    o_ref[...] = jax.nn.sigmoid(pb_ref[0] + pb_ref[1] + pc_ref[0] + pc_ref[1])


def _tc_merge(pb, pc):
    bm = 1000
    return pl.pallas_call(
        _merge_body,
        grid=(N // bm,),
        in_specs=[
            pl.BlockSpec((2, bm, D), lambda i: (0, i, 0)),
            pl.BlockSpec((2, bm, D), lambda i: (0, i, 0)),
        ],
        out_specs=pl.BlockSpec((bm, D), lambda i: (i, 0)),
        out_shape=jax.ShapeDtypeStruct((N, D), _f32),
    )(pb, pc)


_MESH = plsc.VectorSubcoreMesh(core_axis_name="c", subcore_axis_name="s")


def _sc_scatter_body(table_hbm, gidx_hbm, sidx_hbm, zeros_hbm, out_hbm,
                     gv, sv, buf, acc, sem):
    cid = lax.axis_index("c")
    sid = lax.axis_index("s")
    wid = cid * NS + sid
    pltpu.sync_copy(zeros_hbm, acc.at[pl.ds(sid * 640, 640)])
    plsc.subcore_barrier()
    pltpu.sync_copy(gidx_hbm.at[wid], gv)
    pltpu.sync_copy(sidx_hbm.at[wid], sv)

    def chunk(j, carry):
        pltpu.async_copy(table_hbm.at[gv.at[j]], buf, sem).wait()
        pltpu.sync_copy(buf, acc.at[sv.at[j]], add=True)
        return carry

    lax.fori_loop(0, NCH, chunk, 0)
    plsc.subcore_barrier()
    pltpu.sync_copy(acc.at[pl.ds(sid * 640, 640)],
                    out_hbm.at[cid, pl.ds(sid * 640, 640)])


def _sc_scatter(table, gidx, sidx, zeros):
    k = pl.kernel(
        _sc_scatter_body,
        out_type=jax.ShapeDtypeStruct((NC, NACC, D), _f32),
        mesh=_MESH,
        scratch_types=[
            pltpu.VMEM((NCH, CH), _i32),
            pltpu.VMEM((NCH, CH), _i32),
            pltpu.VMEM((CH, D), _f32),
            pltpu.VMEM_SHARED((NACC, D), _f32),
            pltpu.SemaphoreType.DMA,
        ],
    )
    return k(table, gidx, sidx, zeros)


def _sc_edge_body(table_hbm, gidx_hbm, dloc_hbm, zeros_hbm, out_hbm,
                  gv, sv, buf, acc, sem):
    cid = lax.axis_index("c")
    sid = lax.axis_index("s")

    def batch(bi, carry):
        b = bi * NC + cid
        pltpu.sync_copy(zeros_hbm.at[pl.ds(0, 512)],
                        acc.at[pl.ds(sid * 512, 512)])
        plsc.subcore_barrier()
        pltpu.sync_copy(gidx_hbm.at[b, sid], gv)
        pltpu.sync_copy(dloc_hbm.at[b, sid], sv)

        def chunk(j, c2):
            pltpu.async_copy(table_hbm.at[gv.at[j]], buf, sem).wait()
            pltpu.sync_copy(buf, acc.at[sv.at[j]], add=True)
            return c2

        lax.fori_loop(0, ECH, chunk, 0)
        plsc.subcore_barrier()
        pltpu.sync_copy(acc.at[pl.ds(sid * 512, 512)],
                        out_hbm.at[pl.ds(b * EB + sid * 512, 512)])
        return carry

    lax.fori_loop(0, NB // NC, batch, 0)


def _sc_edge(table, gidx, dloc, zeros):
    k = pl.kernel(
        _sc_edge_body,
        out_type=jax.ShapeDtypeStruct((EP, D), _f32),
        mesh=_MESH,
        scratch_types=[
            pltpu.VMEM((ECH, CH), _i32),
            pltpu.VMEM((ECH, CH), _i32),
            pltpu.VMEM((CH, D), _f32),
            pltpu.VMEM_SHARED((EB, D), _f32),
            pltpu.SemaphoreType.DMA,
        ],
    )
    return k(table, gidx, dloc, zeros)


def kernel(x, adj_row, adj_col, inc_node, inc_edge, W1, W2, W3, W4):
    zeros = jnp.zeros((640, D), _f32)

    adj_g = _pad_to(adj_col, NNZ_PAD, 0).reshape(NT, NCH, CH)
    adj_s = _pad_to(adj_row, NNZ_PAD, TRASH).reshape(NT, NCH, CH)
    inc_g = _pad_to(inc_node, 2 * EP, 0).reshape(NB, NS, ECH, CH)
    dloc = ((jnp.arange(2 * EP, dtype=_i32) // 2) % EB).reshape(NB, NS, ECH, CH)
    incf_g = _pad_to(inc_edge, NNZ_PAD, 0).reshape(NT, NCH, CH)
    incf_s = _pad_to(inc_node, NNZ_PAD, TRASH).reshape(NT, NCH, CH)

    y1, y2 = _tc_mm2(x, W1, W2)
    pA = _sc_scatter(y1, adj_g, adj_s, zeros)
    e1 = _sc_edge(y2, inc_g, dloc, zeros)
    z3 = _tc_sig2mm(pA, W3)
    z4 = _tc_sigmm(e1, W4)
    pB = _sc_scatter(z3, adj_g, adj_s, zeros)
    pC = _sc_scatter(z4, incf_g, incf_s, zeros)
    return _tc_merge(pB, pC)
